# Initial kernel scaffold; baseline (speedup 1.0000x reference)
import jax
import jax.numpy as jnp
from jax.experimental import pallas as pl


def kernel(entity_embds, rel_embds):
    raise NotImplementedError("write your pallas kernel here")


# speedup vs baseline: 1.0126x; 1.0126x over previous
import jax
import jax.numpy as jnp
from jax.experimental import pallas as pl

NUM_ENTITIES = 1000000
EMB_DIM = 64
BLOCK_ROWS = 8000


def _normalize_block(ent_ref, out_ref):
    i = pl.program_id(0)
    x = ent_ref[...]
    ss = jnp.sum(x * x, axis=1, keepdims=True)
    inv = 1.0 / jnp.sqrt(ss)
    row = i * BLOCK_ROWS + jax.lax.broadcasted_iota(jnp.int32, (BLOCK_ROWS, 1), 0)
    scale = jnp.where(row == NUM_ENTITIES - 1, 1.0, inv)
    out_ref[...] = x * scale


def kernel(entity_embds, rel_embds):
    grid = NUM_ENTITIES // BLOCK_ROWS
    ent_out = pl.pallas_call(
        _normalize_block,
        grid=(grid,),
        in_specs=[pl.BlockSpec((BLOCK_ROWS, EMB_DIM), lambda i: (i, 0))],
        out_specs=pl.BlockSpec((BLOCK_ROWS, EMB_DIM), lambda i: (i, 0)),
        out_shape=jax.ShapeDtypeStruct((NUM_ENTITIES, EMB_DIM), jnp.float32),
    )(entity_embds)
    return (ent_out, rel_embds)
